# baseline (speedup 1.0000x reference)
import functools

import jax
import jax.numpy as jnp
from jax import lax
from jax.experimental import pallas as pl
from jax.experimental.pallas import tpu as pltpu
from jax.experimental.pallas import tpu_sc as plsc

_TB = 4096


def _router_block(x_ref, w_ref, vals_ref, idx_ref):
    l = jax.lax.dot_general(
        x_ref[0], w_ref[...], (((1,), (1,)), ((), ())),
        preferred_element_type=jnp.float32,
    )
    iota = jax.lax.broadcasted_iota(jnp.int32, l.shape, 1).astype(jnp.float32)
    m1 = jnp.max(l, axis=1, keepdims=True)
    idx1 = jnp.min(jnp.where(l == m1, iota, 64.0), axis=1, keepdims=True)
    hit1 = iota == idx1
    l2 = jnp.where(hit1, -jnp.inf, l)
    m2 = jnp.max(l2, axis=1, keepdims=True)
    idx2 = jnp.min(jnp.where(l2 == m2, iota, 64.0), axis=1, keepdims=True)
    e = jnp.exp(l - m1)
    sinv = 1.0 / jnp.sum(e, axis=1, keepdims=True)
    e1 = jnp.max(jnp.where(hit1, e, 0.0), axis=1, keepdims=True)
    e2 = jnp.max(jnp.where(iota == idx2, e, 0.0), axis=1, keepdims=True)
    vals_ref[0] = jnp.concatenate([e1, e2], axis=1) * sinv
    idx_ref[0] = jnp.concatenate([idx1, idx2], axis=1).astype(jnp.int32)


def _tc_topk(x, W):
    B, T, C = x.shape
    E = W.shape[0]
    grid = (B, T // _TB)
    vals, idx = pl.pallas_call(
        _router_block,
        grid=grid,
        in_specs=[
            pl.BlockSpec((1, _TB, C), lambda b, i: (b, i, 0)),
            pl.BlockSpec((E, C), lambda b, i: (0, 0)),
        ],
        out_specs=[
            pl.BlockSpec((1, _TB, 2), lambda b, i: (b, i, 0)),
            pl.BlockSpec((1, _TB, 2), lambda b, i: (b, i, 0)),
        ],
        out_shape=[
            jax.ShapeDtypeStruct((B, T, 2), jnp.float32),
            jax.ShapeDtypeStruct((B, T, 2), jnp.int32),
        ],
        compiler_params=pltpu.CompilerParams(
            dimension_semantics=("arbitrary", "arbitrary"),
        ),
    )(x, W)
    return vals, idx


def _sc_dispatch(valsf, idxf, N, E):
    info = plsc.get_sparse_core_info()
    NC, NS, L = info.num_cores, info.num_subcores, info.num_lanes
    NW = NC * NS
    tok = N // NW
    mesh = plsc.VectorSubcoreMesh(core_axis_name="c", subcore_axis_name="s")

    @functools.partial(
        pl.kernel,
        out_type=jax.ShapeDtypeStruct((N * E,), jnp.float32),
        mesh=mesh,
        scratch_types=[
            pltpu.VMEM((tok * 2,), jnp.float32),
            pltpu.VMEM((tok * 2,), jnp.int32),
            pltpu.VMEM((tok * E,), jnp.float32),
        ],
        compiler_params=pltpu.CompilerParams(needs_layout_passes=False),
    )
    def sc_kernel(vals_hbm, idx_hbm, out_hbm, vals_v, idx_v, out_v):
        wid = lax.axis_index("s") * NC + lax.axis_index("c")
        base = wid * tok
        pltpu.sync_copy(vals_hbm.at[pl.ds(base * 2, tok * 2)], vals_v)
        pltpu.sync_copy(idx_hbm.at[pl.ds(base * 2, tok * 2)], idx_v)
        zero16 = jnp.zeros((L,), jnp.float32)

        def zbody(i, carry):
            out_v[pl.ds(i * (4 * L), L)] = zero16
            out_v[pl.ds(i * (4 * L) + L, L)] = zero16
            out_v[pl.ds(i * (4 * L) + 2 * L, L)] = zero16
            out_v[pl.ds(i * (4 * L) + 3 * L, L)] = zero16
            return carry

        lax.fori_loop(0, tok * E // (4 * L), zbody, 0)

        iota16 = lax.iota(jnp.int32, L)

        def sbody(j, carry):
            q = j * L
            e16 = idx_v[pl.ds(q, L)]
            v16 = vals_v[pl.ds(q, L)]
            tok_local = (q + iota16) >> 1
            addr = (tok_local << 6) + e16
            plsc.store_scatter(out_v, [addr], v16)
            return carry

        lax.fori_loop(0, tok * 2 // L, sbody, 0)
        pltpu.sync_copy(out_v, out_hbm.at[pl.ds(base * E, tok * E)])

    return sc_kernel(valsf, idxf)


@jax.jit
def kernel(x, W):
    B, T, C = x.shape
    E = W.shape[0]
    N = B * T
    vals, idx = _tc_topk(x, W)
    out = _sc_dispatch(vals.reshape(N * 2), idx.reshape(N * 2), N, E)
    return out.reshape(B, T, E), idx


# speedup vs baseline: 1.7323x; 1.7323x over previous
import functools

import jax
import jax.numpy as jnp
from jax.experimental import pallas as pl
from jax.experimental.pallas import tpu as pltpu

_TB = 4096


def _router_block(x_ref, w_ref, out_ref, idx_ref):
    l = jax.lax.dot_general(
        x_ref[0], w_ref[...], (((1,), (1,)), ((), ())),
        preferred_element_type=jnp.float32,
    )
    iota = jax.lax.broadcasted_iota(jnp.int32, l.shape, 1).astype(jnp.float32)
    m1 = jnp.max(l, axis=1, keepdims=True)
    idx1 = jnp.min(jnp.where(l == m1, iota, 64.0), axis=1, keepdims=True)
    hit1 = iota == idx1
    l2 = jnp.where(hit1, -jnp.inf, l)
    m2 = jnp.max(l2, axis=1, keepdims=True)
    idx2 = jnp.min(jnp.where(l2 == m2, iota, 64.0), axis=1, keepdims=True)
    e = jnp.exp(l - m1)
    sinv = 1.0 / jnp.sum(e, axis=1, keepdims=True)
    out_ref[0] = jnp.where(hit1 | (iota == idx2), e * sinv, 0.0)
    idx_ref[0] = jnp.concatenate([idx1, idx2], axis=1).astype(jnp.int32)


@jax.jit
def kernel(x, W):
    B, T, C = x.shape
    E = W.shape[0]
    grid = (B, T // _TB)
    out, idx = pl.pallas_call(
        _router_block,
        grid=grid,
        in_specs=[
            pl.BlockSpec((1, _TB, C), lambda b, i: (b, i, 0)),
            pl.BlockSpec((E, C), lambda b, i: (0, 0)),
        ],
        out_specs=[
            pl.BlockSpec((1, _TB, E), lambda b, i: (b, i, 0)),
            pl.BlockSpec((1, _TB, 2), lambda b, i: (b, i, 0)),
        ],
        out_shape=[
            jax.ShapeDtypeStruct((B, T, E), jnp.float32),
            jax.ShapeDtypeStruct((B, T, 2), jnp.int32),
        ],
        compiler_params=pltpu.CompilerParams(
            dimension_semantics=("arbitrary", "arbitrary"),
        ),
    )(x, W)
    return out, idx
